# baseline (speedup 1.0000x reference)
import jax
import jax.numpy as jnp
from jax.experimental import pallas as pl
from jax.experimental.pallas import tpu as pltpu

_CHANS = (3, 4, 8, 16, 32, 64, 128)


def _im2col(x, ho):
    t, b, h, w, c = x.shape
    xp = jnp.pad(x, ((0, 0), (0, 0), (1, 1), (1, 1), (0, 0)))
    parts = [xp[:, :, di:di + 2 * ho:2, dj:dj + 2 * ho:2, :]
             for di in range(4) for dj in range(4)]
    p = jnp.stack(parts, axis=4)
    return p.reshape(t, b * ho * ho, 16 * c)


def _dot3(a, b):
    f32 = jnp.float32
    ah = a.astype(jnp.bfloat16)
    al = (a - ah.astype(f32)).astype(jnp.bfloat16)
    bh = b.astype(jnp.bfloat16)
    bl = (b - bh.astype(f32)).astype(jnp.bfloat16)
    return (jnp.dot(ah, bh, preferred_element_type=f32)
            + jnp.dot(ah, bl, preferred_element_type=f32)
            + jnp.dot(al, bh, preferred_element_type=f32))


def _matmul_relu_kernel(p_ref, w_ref, b_ref, o_ref):
    y = _dot3(p_ref[0], w_ref[0])
    o_ref[0] = jnp.maximum(y + b_ref[0], 0.0)


def _conv_layer(p, wmat, bias, mblk):
    t, r, k = p.shape
    cout = wmat.shape[2]
    return pl.pallas_call(
        _matmul_relu_kernel,
        grid=(r // mblk, t),
        in_specs=[
            pl.BlockSpec((1, mblk, k), lambda m, tt: (tt, m, 0)),
            pl.BlockSpec((1, k, cout), lambda m, tt: (tt, 0, 0)),
            pl.BlockSpec((1, 1, cout), lambda m, tt: (tt, 0, 0)),
        ],
        out_specs=pl.BlockSpec((1, mblk, cout), lambda m, tt: (tt, m, 0)),
        out_shape=jax.ShapeDtypeStruct((t, r, cout), jnp.float32),
        compiler_params=pltpu.CompilerParams(
            dimension_semantics=("parallel", "arbitrary")),
    )(p, wmat, bias)


def _tail_kernel(e_ref, mw1, mb1, mw2, mb2, mw3, mb3,
                 wihg, whhg, blg, hw1, hb1, hw2, hb2, hw3, hb3, out_ref):
    f32 = jnp.float32
    n = e_ref.shape[1]
    feats = []
    for t in range(3):
        h = jnp.maximum(_dot3(e_ref[t], mw1[...]) + mb1[...], 0.0)
        h = jnp.maximum(_dot3(h, mw2[...]) + mb2[...], 0.0)
        feats.append(_dot3(h, mw3[...]) + mb3[...])
    hcur = jnp.zeros((n, 20), f32)
    ccur = jnp.zeros((n, 20), f32)
    hs = []
    for t in range(3):
        pre = [_dot3(feats[t], wihg[g]) + _dot3(hcur, whhg[g])
               + blg[g] for g in range(4)]
        ig = jax.nn.sigmoid(pre[0])
        fg = jax.nn.sigmoid(pre[1])
        gg = jnp.tanh(pre[2])
        og = jax.nn.sigmoid(pre[3])
        ccur = fg * ccur + ig * gg
        hcur = og * jnp.tanh(ccur)
        hs.append(hcur)
    rows = jnp.stack(hs, axis=0).reshape(3 * n, 20)
    y = jnp.maximum(_dot3(rows, hw1[...]) + hb1[...], 0.0)
    y = jnp.maximum(_dot3(y, hw2[...]) + hb2[...], 0.0)
    y = _dot3(y, hw3[...]) + hb3[...]
    out_ref[...] = y.reshape(3, n, 20)


def _full_spec(a):
    nd = len(a.shape)
    return pl.BlockSpec(a.shape, lambda i, _nd=nd: (0,) * _nd)


def kernel(img_seq, enc_w0, enc_b0, enc_w1, enc_b1, enc_w2, enc_b2,
           enc_w3, enc_b3, enc_w4, enc_b4, enc_w5, enc_b5,
           mlp_w1, mlp_b1, mlp_w2, mlp_b2, mlp_w3, mlp_b3,
           w_ih, w_hh, b_ih, b_hh,
           head_w1, head_b1, head_w2, head_b2, head_w3, head_b3):
    T, B = img_seq.shape[0], img_seq.shape[1]
    spatial = (32, 16, 8, 4, 2, 1)

    x = img_seq.transpose(0, 1, 3, 4, 2)
    enc_ws = (enc_w0, enc_w1, enc_w2, enc_w3, enc_w4, enc_w5)
    enc_bs = (enc_b0, enc_b1, enc_b2, enc_b3, enc_b4, enc_b5)
    for l in range(6):
        ho = spatial[l]
        cin, cout = _CHANS[l], _CHANS[l + 1]
        p = _im2col(x, ho)
        w = enc_ws[l].transpose(0, 3, 4, 2, 1).reshape(T, 16 * cin, cout)
        bias = enc_bs[l].reshape(T, 1, cout)
        r = p.shape[1]
        k = p.shape[2]
        mblk = min(r, 8192, max(512, 2097152 // (4 * k)))
        y = _conv_layer(p, w, bias, mblk)
        x = y.reshape(T, B, ho, ho, cout)

    enc = x.reshape(T, B, 128)

    mw1, mw2, mw3 = mlp_w1.T, mlp_w2.T, mlp_w3.T
    mb1, mb2, mb3 = (mlp_b1.reshape(1, -1), mlp_b2.reshape(1, -1),
                     mlp_b3.reshape(1, -1))
    wihg = jnp.stack([w_ih.T[:, 20 * g:20 * (g + 1)] for g in range(4)])
    whhg = jnp.stack([w_hh.T[:, 20 * g:20 * (g + 1)] for g in range(4)])
    blg = (b_ih + b_hh).reshape(4, 1, 20)
    K = head_w1.shape[0]
    hw1 = head_w1.transpose(2, 0, 1).reshape(20, K * 32)
    hb1 = head_b1.reshape(1, K * 32)
    hw2 = jax.scipy.linalg.block_diag(*[head_w2[k].T for k in range(K)])
    hb2 = head_b2.reshape(1, K * 32)
    hw3 = jax.scipy.linalg.block_diag(*[head_w3[k].T for k in range(K)])
    hb3 = head_b3.reshape(1, K)

    operands = [enc, mw1, mb1, mw2, mb2, mw3, mb3, wihg, whhg, blg,
                hw1, hb1, hw2, hb2, hw3, hb3]
    nblk = min(512, B)
    in_specs = [pl.BlockSpec((T, nblk, 128), lambda i: (0, i, 0))]
    in_specs += [_full_spec(a) for a in operands[1:]]
    out = pl.pallas_call(
        _tail_kernel,
        grid=(B // nblk,),
        in_specs=in_specs,
        out_specs=pl.BlockSpec((3, nblk, 20), lambda i: (0, i, 0)),
        out_shape=jax.ShapeDtypeStruct((3, B, 20), jnp.float32),
        compiler_params=pltpu.CompilerParams(
            dimension_semantics=("parallel",)),
    )(*operands)

    return jnp.transpose(out, (2, 0, 1))[..., None]
